# baseline (speedup 1.0000x reference)
import jax
import jax.numpy as jnp
from jax.experimental import pallas as pl
from jax.experimental.pallas import tpu as pltpu

C = 128
H = C // 2
EPS = 1e-5
GATE = 1.0 / 1.2 ** 2


def _tile(n, cap=256):
    for t in (cap, 128, 64, 32, 16):
        if t <= n and n % t == 0:
            return t
    return n


def _dnt(a, b):
    return jax.lax.dot_general(a, b, (((1,), (1,)), ((), ())),
                               preferred_element_type=jnp.float32)


def _bn_relu(y, g, b):
    m = y.shape[0]
    ones = jnp.ones((1, m), jnp.float32)
    inv_m = 1.0 / m
    mu = jnp.dot(ones, y, preferred_element_type=jnp.float32) * inv_m
    d = y - mu
    var = jnp.dot(ones, d * d, preferred_element_type=jnp.float32) * inv_m
    return jnp.maximum(d * (g * jax.lax.rsqrt(var + EPS)) + b, 0.0)


def _dense(a, w_ref, b_ref):
    return jnp.dot(a.astype(jnp.bfloat16), w_ref[...],
                   preferred_element_type=jnp.float32) + b_ref[...]


def _proj_body(x_ref, wc_ref, bc_ref, gc_ref, bec_ref, wqkv_ref, bqkv_ref,
               feat_ref, q_ref, k_ref, v_ref):
    feat = _bn_relu(_dense(x_ref[...], wc_ref, bc_ref), gc_ref[...], bec_ref[...])
    feat_ref[...] = feat
    qkv = _dense(feat, wqkv_ref, bqkv_ref)
    q_ref[...] = qkv[:, :C].astype(q_ref.dtype)
    k_ref[...] = qkv[:, C:2 * C].astype(k_ref.dtype)
    v_ref[...] = qkv[:, 2 * C:].astype(v_ref.dtype)


def _attn_body(sq_ref, sk_ref, tq_ref, tk_ref, q_ref, k_ref, v_ref, o_ref):
    def dist(pq, pk):
        rq = jnp.sum(pq * pq, axis=-1, keepdims=True)
        rk = _dnt(jnp.ones((1, pk.shape[-1]), jnp.float32), pk * pk)
        return jnp.sqrt(jnp.maximum(rq + rk - 2.0 * _dnt(pq, pk), 0.0))

    dd = dist(sq_ref[0], sk_ref[0]) - dist(tq_ref[0], tk_ref[0])
    gate = jnp.maximum(1.0 - dd * dd * GATE, 0.0).astype(jnp.bfloat16).astype(jnp.float32)

    logits = gate * _dnt(q_ref[0], k_ref[0])
    mx = jnp.max(logits, axis=-1, keepdims=True)
    e = jnp.exp(logits - mx)
    w = e * pl.reciprocal(jnp.sum(e, axis=-1, keepdims=True), approx=True)
    o_ref[0] = jnp.dot(w.astype(jnp.bfloat16), v_ref[0],
                       preferred_element_type=jnp.float32).astype(o_ref.dtype)


def _fc_proj_body(msg_ref, feat_ref,
                  w1_ref, b1_ref, g1_ref, be1_ref,
                  w2_ref, b2_ref, g2_ref, be2_ref,
                  w3_ref, b3_ref,
                  wc_ref, bc_ref, gc_ref, bec_ref, wqkv_ref, bqkv_ref,
                  feat_out_ref, q_ref, k_ref, v_ref):
    m1 = _bn_relu(_dense(msg_ref[...], w1_ref, b1_ref), g1_ref[...], be1_ref[...])
    m2 = _bn_relu(_dense(m1, w2_ref, b2_ref), g2_ref[...], be2_ref[...])
    res = feat_ref[...] + _dense(m2, w3_ref, b3_ref)
    feat = _bn_relu(_dense(res, wc_ref, bc_ref), gc_ref[...], bec_ref[...])
    feat_out_ref[...] = feat
    qkv = _dense(feat, wqkv_ref, bqkv_ref)
    q_ref[...] = qkv[:, :C].astype(q_ref.dtype)
    k_ref[...] = qkv[:, C:2 * C].astype(k_ref.dtype)
    v_ref[...] = qkv[:, 2 * C:].astype(v_ref.dtype)


def _fc_body(msg_ref, feat_ref,
             w1_ref, b1_ref, g1_ref, be1_ref,
             w2_ref, b2_ref, g2_ref, be2_ref,
             w3_ref, b3_ref, out_ref):
    m1 = _bn_relu(_dense(msg_ref[...], w1_ref, b1_ref), g1_ref[...], be1_ref[...])
    m2 = _bn_relu(_dense(m1, w2_ref, b2_ref), g2_ref[...], be2_ref[...])
    out_ref[...] = feat_ref[...] + _dense(m2, w3_ref, b3_ref)


def _wT(w):
    return jnp.transpose(w).astype(jnp.bfloat16)


def _row(v):
    return v.reshape(1, -1)


def _attention(q, k, v, src_p, tgt_p, bs, n, tq):
    qs_c = pl.BlockSpec((1, tq, 8), lambda b, i: (b, i, 0))
    ks_c = pl.BlockSpec((1, n, 8), lambda b, i: (b, 0, 0))
    msg = pl.pallas_call(
        _attn_body,
        out_shape=jax.ShapeDtypeStruct((bs, n, C), jnp.bfloat16),
        grid=(bs, n // tq),
        in_specs=[qs_c, ks_c, qs_c, ks_c,
                  pl.BlockSpec((1, tq, C), lambda b, i: (b, i, 0)),
                  pl.BlockSpec((1, n, C), lambda b, i: (b, 0, 0)),
                  pl.BlockSpec((1, n, C), lambda b, i: (b, 0, 0))],
        out_specs=pl.BlockSpec((1, tq, C), lambda b, i: (b, i, 0)),
        compiler_params=pltpu.CompilerParams(
            dimension_semantics=("parallel", "parallel"),
            vmem_limit_bytes=48 << 20),
    )(src_p, src_p, tgt_p, tgt_p, q, k, v)
    return msg.reshape(bs * n, C)


def _fc_args(lp):
    return (_wT(lp['w1']), _row(lp['b1']), _row(lp['g1']), _row(lp['be1']),
            _wT(lp['w2']), _row(lp['b2']), _row(lp['g2']), _row(lp['be2']),
            _wT(lp['w3']), _row(lp['b3']))


def _proj_args(wc, bc, lp):
    inv = 1.0 / float(C) ** 0.5
    wqkv = jnp.transpose(jnp.concatenate(
        [lp['wq'] * inv, lp['wk'], lp['wv']], axis=0)).astype(jnp.bfloat16)
    bqkv = jnp.concatenate([lp['bq'] * inv, lp['bk'], lp['bv']]).reshape(1, 3 * C)
    return (_wT(wc), _row(bc), _row(lp['gc']), _row(lp['bec']), wqkv, bqkv)


def kernel(w0, b0,
           l0_wc, l0_bc, l0_gc, l0_bec,
           l0_wq, l0_bq, l0_wk, l0_bk, l0_wv, l0_bv,
           l0_w1, l0_b1, l0_g1, l0_be1, l0_w2, l0_b2, l0_g2, l0_be2, l0_w3, l0_b3,
           l1_wc, l1_bc, l1_gc, l1_bec,
           l1_wq, l1_bq, l1_wk, l1_bk, l1_wv, l1_bv,
           l1_w1, l1_b1, l1_g1, l1_be1, l1_w2, l1_b2, l1_g2, l1_be2, l1_w3, l1_b3,
           src_keypts, tgt_keypts_all):
    l0 = {'wc': l0_wc, 'bc': l0_bc, 'gc': l0_gc, 'bec': l0_bec,
          'wq': l0_wq, 'bq': l0_bq, 'wk': l0_wk, 'bk': l0_bk, 'wv': l0_wv, 'bv': l0_bv,
          'w1': l0_w1, 'b1': l0_b1, 'g1': l0_g1, 'be1': l0_be1,
          'w2': l0_w2, 'b2': l0_b2, 'g2': l0_g2, 'be2': l0_be2,
          'w3': l0_w3, 'b3': l0_b3}
    l1 = {'wc': l1_wc, 'bc': l1_bc, 'gc': l1_gc, 'bec': l1_bec,
          'wq': l1_wq, 'bq': l1_bq, 'wk': l1_wk, 'bk': l1_bk, 'wv': l1_wv, 'bv': l1_bv,
          'w1': l1_w1, 'b1': l1_b1, 'g1': l1_g1, 'be1': l1_be1,
          'w2': l1_w2, 'b2': l1_b2, 'g2': l1_g2, 'be2': l1_be2,
          'w3': l1_w3, 'b3': l1_b3}

    bs, n, _ = src_keypts.shape
    m = bs * n
    tq = _tile(n)
    tgt = jnp.mean(tgt_keypts_all, axis=2)
    src_p = jnp.pad(src_keypts, ((0, 0), (0, 0), (0, 5)))
    tgt_p = jnp.pad(tgt, ((0, 0), (0, 0), (0, 5)))

    corr = jnp.concatenate([src_keypts, tgt], axis=-1)
    corr = corr - jnp.mean(corr, axis=1, keepdims=True)
    x = jnp.pad(corr.reshape(m, 6), ((0, 0), (0, 2)))

    wc0 = jnp.pad(l0_wc @ w0, ((0, 0), (0, 2)))
    bc0 = l0_wc @ b0 + l0_bc

    s3 = (bs, n, C)
    fshape = jax.ShapeDtypeStruct((m, C), jnp.float32)
    bshape = jax.ShapeDtypeStruct((m, C), jnp.bfloat16)

    feat, q, k, v = pl.pallas_call(
        _proj_body,
        out_shape=(fshape, bshape, bshape, bshape),
        compiler_params=pltpu.CompilerParams(vmem_limit_bytes=64 << 20),
    )(x, *_proj_args(wc0, bc0, l0))
    msg = _attention(q.reshape(s3), k.reshape(s3), v.reshape(s3), src_p, tgt_p, bs, n, tq)

    feat, q, k, v = pl.pallas_call(
        _fc_proj_body,
        out_shape=(fshape, bshape, bshape, bshape),
        compiler_params=pltpu.CompilerParams(vmem_limit_bytes=96 << 20),
    )(msg, feat, *_fc_args(l0), *_proj_args(l1['wc'], l1['bc'], l1))
    msg = _attention(q.reshape(s3), k.reshape(s3), v.reshape(s3), src_p, tgt_p, bs, n, tq)

    out = pl.pallas_call(
        _fc_body,
        out_shape=fshape,
        compiler_params=pltpu.CompilerParams(vmem_limit_bytes=64 << 20),
    )(msg, feat, *_fc_args(l1))

    return jnp.transpose(out.reshape(bs, n, C), (0, 2, 1))


# speedup vs baseline: 1.1453x; 1.1453x over previous
import jax
import jax.numpy as jnp
from jax.experimental import pallas as pl
from jax.experimental.pallas import tpu as pltpu

C = 128
EPS = 1e-5
GK = 1.0 / 1.2 ** 2


def _tile(n, cap=512):
    for t in (cap, 256, 128, 64, 32, 16):
        if t <= n and n % t == 0:
            return t
    return n


def _dnt(a, b):
    return jax.lax.dot_general(a, b, (((1,), (1,)), ((), ())),
                               preferred_element_type=jnp.float32)


def _bn_relu(y, g, b):
    m = y.shape[0]
    ones = jnp.ones((1, m), jnp.float32)
    inv_m = 1.0 / m
    mu = jnp.dot(ones, y, preferred_element_type=jnp.float32) * inv_m
    d = y - mu
    var = jnp.dot(ones, d * d, preferred_element_type=jnp.float32) * inv_m
    return jnp.maximum(d * (g * jax.lax.rsqrt(var + EPS)) + b, 0.0)


def _dense(a, w_ref, b_ref):
    return jnp.dot(a.astype(jnp.bfloat16), w_ref[...],
                   preferred_element_type=jnp.float32) + b_ref[...]


def _proj_body(x_ref, wc_ref, bc_ref, gc_ref, bec_ref, wqkv_ref, bqkv_ref,
               feat_ref, q_ref, k_ref, v_ref):
    feat = _bn_relu(_dense(x_ref[...], wc_ref, bc_ref), gc_ref[...], bec_ref[...])
    feat_ref[...] = feat
    qkv = _dense(feat, wqkv_ref, bqkv_ref)
    q_ref[...] = qkv[:, :C].astype(q_ref.dtype)
    k_ref[...] = qkv[:, C:2 * C].astype(k_ref.dtype)
    v_ref[...] = qkv[:, 2 * C:].astype(v_ref.dtype)


def _dists(aq, ak):
    inner = _dnt(aq, ak)
    rq = jnp.sum(aq * aq, axis=-1, keepdims=True)
    ones = jnp.ones((1, ak.shape[-1]), jnp.float32)
    rk = _dnt(ones, ak * ak)
    return jnp.sqrt(jnp.maximum(rq + rk - 2.0 * inner, 0.0))


def _softmax_message(gate16, q, k, v, o_ref):
    logits = gate16.astype(jnp.float32) * _dnt(q, k)
    mx = jnp.max(logits, axis=-1, keepdims=True)
    e = jnp.exp(logits - mx)
    w = e * pl.reciprocal(jnp.sum(e, axis=-1, keepdims=True), approx=True)
    o_ref[0] = jnp.dot(w.astype(jnp.bfloat16), v,
                       preferred_element_type=jnp.float32).astype(o_ref.dtype)


def _attn_gate_body(sq_ref, sk_ref, tq_ref, tk_ref, q_ref, k_ref, v_ref,
                    o_ref, g_ref):
    compat = _dists(sq_ref[0], sk_ref[0]) - _dists(tq_ref[0], tk_ref[0])
    gate16 = jnp.maximum(1.0 - compat * compat * GK, 0.0).astype(jnp.bfloat16)
    g_ref[0] = gate16
    _softmax_message(gate16, q_ref[0], k_ref[0], v_ref[0], o_ref)


def _attn_reuse_body(g_in_ref, q_ref, k_ref, v_ref, o_ref):
    _softmax_message(g_in_ref[0], q_ref[0], k_ref[0], v_ref[0], o_ref)


def _fc_stack(msg_ref, feat_ref, w1_ref, b1_ref, g1_ref, be1_ref,
              w2_ref, b2_ref, g2_ref, be2_ref, w3_ref, b3_ref):
    m1 = _bn_relu(_dense(msg_ref[...], w1_ref, b1_ref), g1_ref[...], be1_ref[...])
    m2 = _bn_relu(_dense(m1, w2_ref, b2_ref), g2_ref[...], be2_ref[...])
    return feat_ref[...] + _dense(m2, w3_ref, b3_ref)


def _fc_proj_body(msg_ref, feat_ref,
                  w1_ref, b1_ref, g1_ref, be1_ref,
                  w2_ref, b2_ref, g2_ref, be2_ref, w3_ref, b3_ref,
                  wc_ref, bc_ref, gc_ref, bec_ref, wqkv_ref, bqkv_ref,
                  feat_out_ref, q_ref, k_ref, v_ref):
    res = _fc_stack(msg_ref, feat_ref, w1_ref, b1_ref, g1_ref, be1_ref,
                    w2_ref, b2_ref, g2_ref, be2_ref, w3_ref, b3_ref)
    feat = _bn_relu(_dense(res, wc_ref, bc_ref), gc_ref[...], bec_ref[...])
    feat_out_ref[...] = feat
    qkv = _dense(feat, wqkv_ref, bqkv_ref)
    q_ref[...] = qkv[:, :C].astype(q_ref.dtype)
    k_ref[...] = qkv[:, C:2 * C].astype(k_ref.dtype)
    v_ref[...] = qkv[:, 2 * C:].astype(v_ref.dtype)


def _fc_body(msg_ref, feat_ref,
             w1_ref, b1_ref, g1_ref, be1_ref,
             w2_ref, b2_ref, g2_ref, be2_ref, w3_ref, b3_ref, out_ref):
    out_ref[...] = _fc_stack(msg_ref, feat_ref, w1_ref, b1_ref, g1_ref, be1_ref,
                             w2_ref, b2_ref, g2_ref, be2_ref, w3_ref, b3_ref)


def _wT(w):
    return jnp.transpose(w).astype(jnp.bfloat16)


def _row(v):
    return v.reshape(1, -1)


def _attention0(q, k, v, src_p, tgt_p, bs, n, tq):
    qs_c = pl.BlockSpec((1, tq, 8), lambda b, i: (b, i, 0))
    ks_c = pl.BlockSpec((1, n, 8), lambda b, i: (b, 0, 0))
    msg, gate = pl.pallas_call(
        _attn_gate_body,
        out_shape=(jax.ShapeDtypeStruct((bs, n, C), jnp.bfloat16),
                   jax.ShapeDtypeStruct((bs, n, n), jnp.bfloat16)),
        grid=(bs, n // tq),
        in_specs=[qs_c, ks_c, qs_c, ks_c,
                  pl.BlockSpec((1, tq, C), lambda b, i: (b, i, 0)),
                  pl.BlockSpec((1, n, C), lambda b, i: (b, 0, 0)),
                  pl.BlockSpec((1, n, C), lambda b, i: (b, 0, 0))],
        out_specs=(pl.BlockSpec((1, tq, C), lambda b, i: (b, i, 0)),
                   pl.BlockSpec((1, tq, n), lambda b, i: (b, i, 0))),
        compiler_params=pltpu.CompilerParams(
            dimension_semantics=("parallel", "parallel"),
            vmem_limit_bytes=64 << 20),
    )(src_p, src_p, tgt_p, tgt_p, q, k, v)
    return msg.reshape(bs * n, C), gate


def _attention1(q, k, v, gate, bs, n, tq):
    msg = pl.pallas_call(
        _attn_reuse_body,
        out_shape=jax.ShapeDtypeStruct((bs, n, C), jnp.bfloat16),
        grid=(bs, n // tq),
        in_specs=[pl.BlockSpec((1, tq, n), lambda b, i: (b, i, 0)),
                  pl.BlockSpec((1, tq, C), lambda b, i: (b, i, 0)),
                  pl.BlockSpec((1, n, C), lambda b, i: (b, 0, 0)),
                  pl.BlockSpec((1, n, C), lambda b, i: (b, 0, 0))],
        out_specs=pl.BlockSpec((1, tq, C), lambda b, i: (b, i, 0)),
        compiler_params=pltpu.CompilerParams(
            dimension_semantics=("parallel", "parallel"),
            vmem_limit_bytes=64 << 20),
    )(gate, q, k, v)
    return msg.reshape(bs * n, C)


def _fc_args(w1, b1, g1, be1, w2, b2, g2, be2, w3, b3):
    return (_wT(w1), _row(b1), _row(g1), _row(be1),
            _wT(w2), _row(b2), _row(g2), _row(be2),
            _wT(w3), _row(b3))


def _proj_args(wc, bc, gc, bec, wq, bq, wk, bk, wv, bv):
    inv = 1.0 / float(C) ** 0.5
    wqkv = jnp.transpose(jnp.concatenate(
        [wq * inv, wk, wv], axis=0)).astype(jnp.bfloat16)
    bqkv = jnp.concatenate([bq * inv, bk, bv]).reshape(1, 3 * C)
    return (_wT(wc), _row(bc), _row(gc), _row(bec), wqkv, bqkv)


def kernel(w0, b0,
           l0_wc, l0_bc, l0_gc, l0_bec,
           l0_wq, l0_bq, l0_wk, l0_bk, l0_wv, l0_bv,
           l0_w1, l0_b1, l0_g1, l0_be1, l0_w2, l0_b2, l0_g2, l0_be2, l0_w3, l0_b3,
           l1_wc, l1_bc, l1_gc, l1_bec,
           l1_wq, l1_bq, l1_wk, l1_bk, l1_wv, l1_bv,
           l1_w1, l1_b1, l1_g1, l1_be1, l1_w2, l1_b2, l1_g2, l1_be2, l1_w3, l1_b3,
           src_keypts, tgt_keypts_all):
    bs, n, _ = src_keypts.shape
    m = bs * n
    tq = _tile(n)
    tgt = jnp.mean(tgt_keypts_all, axis=2)
    src_p = jnp.pad(src_keypts, ((0, 0), (0, 0), (0, 5)))
    tgt_p = jnp.pad(tgt, ((0, 0), (0, 0), (0, 5)))

    corr = jnp.concatenate([src_keypts, tgt], axis=-1)
    corr = corr - jnp.mean(corr, axis=1, keepdims=True)
    x = jnp.pad(corr.reshape(m, 6), ((0, 0), (0, 2)))

    wc0 = jnp.pad(l0_wc @ w0, ((0, 0), (0, 2)))
    bc0 = l0_wc @ b0 + l0_bc

    s3 = (bs, n, C)
    fshape = jax.ShapeDtypeStruct((m, C), jnp.float32)
    bshape = jax.ShapeDtypeStruct((m, C), jnp.bfloat16)

    feat, q, k, v = pl.pallas_call(
        _proj_body,
        out_shape=(fshape, bshape, bshape, bshape),
        compiler_params=pltpu.CompilerParams(vmem_limit_bytes=64 << 20),
    )(x, *_proj_args(wc0, bc0, l0_gc, l0_bec, l0_wq, l0_bq, l0_wk, l0_bk, l0_wv, l0_bv))
    msg, gate = _attention0(q.reshape(s3), k.reshape(s3), v.reshape(s3),
                            src_p, tgt_p, bs, n, tq)

    feat, q, k, v = pl.pallas_call(
        _fc_proj_body,
        out_shape=(fshape, bshape, bshape, bshape),
        compiler_params=pltpu.CompilerParams(vmem_limit_bytes=96 << 20),
    )(msg, feat,
      *_fc_args(l0_w1, l0_b1, l0_g1, l0_be1, l0_w2, l0_b2, l0_g2, l0_be2, l0_w3, l0_b3),
      *_proj_args(l1_wc, l1_bc, l1_gc, l1_bec, l1_wq, l1_bq, l1_wk, l1_bk, l1_wv, l1_bv))
    msg = _attention1(q.reshape(s3), k.reshape(s3), v.reshape(s3), gate, bs, n, tq)

    out = pl.pallas_call(
        _fc_body,
        out_shape=fshape,
        compiler_params=pltpu.CompilerParams(vmem_limit_bytes=64 << 20),
    )(msg, feat,
      *_fc_args(l1_w1, l1_b1, l1_g1, l1_be1, l1_w2, l1_b2, l1_g2, l1_be2, l1_w3, l1_b3))

    return jnp.transpose(out.reshape(bs, n, C), (0, 2, 1))


# speedup vs baseline: 1.2439x; 1.0861x over previous
import jax
import jax.numpy as jnp
from jax.experimental import pallas as pl
from jax.experimental.pallas import tpu as pltpu

C = 128
EPS = 1e-5
GK = 1.0 / 1.2 ** 2
INV = 1.0 / float(C) ** 0.5
BF = jnp.bfloat16
F32 = jnp.float32


def _tile(n, cap=512):
    for t in (cap, 256, 128, 64, 32, 16):
        if t <= n and n % t == 0:
            return t
    return n


def _dnt(a, b):
    return jax.lax.dot_general(a, b, (((1,), (1,)), ((), ())),
                               preferred_element_type=F32)


def _bn_relu(y, g_ref, b_ref):
    m = y.shape[0]
    ones = jnp.ones((1, m), F32)
    inv_m = 1.0 / m
    mu = jnp.dot(ones, y, preferred_element_type=F32) * inv_m
    d = y - mu
    var = jnp.dot(ones, d * d, preferred_element_type=F32) * inv_m
    return jnp.maximum(d * (g_ref[...] * jax.lax.rsqrt(var + EPS)) + b_ref[...], 0.0)


def _dense_raw(a, w_ref, b_ref):
    return _dnt(a.astype(BF), w_ref[...].astype(BF)) + b_ref[...]


def _qkv_out(feat, wq_ref, bq_ref, wk_ref, bk_ref, wv_ref, bv_ref,
             q_ref, k_ref, v_ref):
    f16 = feat.astype(BF)
    q_ref[...] = (_dnt(f16, (wq_ref[...] * INV).astype(BF))
                  + bq_ref[...] * INV).astype(BF)
    k_ref[...] = (_dnt(f16, wk_ref[...].astype(BF)) + bk_ref[...]).astype(BF)
    v_ref[...] = (_dnt(f16, wv_ref[...].astype(BF)) + bv_ref[...]).astype(BF)


def _proj0_body(x_ref, w0p_ref, b0_ref, wc_ref, bc_ref, gc_ref, bec_ref,
                wq_ref, bq_ref, wk_ref, bk_ref, wv_ref, bv_ref,
                feat_ref, q_ref, k_ref, v_ref):
    fold = jnp.dot(wc_ref[...], w0p_ref[...], preferred_element_type=F32)
    bias = _dnt(b0_ref[...], wc_ref[...]) + bc_ref[...]
    y = _dnt(x_ref[...].astype(BF), fold.astype(BF)) + bias
    feat = _bn_relu(y, gc_ref, bec_ref)
    feat_ref[...] = feat
    _qkv_out(feat, wq_ref, bq_ref, wk_ref, bk_ref, wv_ref, bv_ref, q_ref, k_ref, v_ref)


def _dists(aq, ak):
    inner = _dnt(aq, ak)
    rq = jnp.sum(aq * aq, axis=-1, keepdims=True)
    ones = jnp.ones((1, ak.shape[-1]), F32)
    rk = _dnt(ones, ak * ak)
    return jnp.sqrt(jnp.maximum(rq + rk - 2.0 * inner, 0.0))


def _softmax_message(gate16, q, k, v, o_ref):
    logits = gate16.astype(F32) * _dnt(q, k)
    mx = jnp.max(logits, axis=-1, keepdims=True)
    e = jnp.exp(logits - mx)
    w = e * pl.reciprocal(jnp.sum(e, axis=-1, keepdims=True), approx=True)
    o_ref[0] = jnp.dot(w.astype(BF), v, preferred_element_type=F32).astype(o_ref.dtype)


def _attn_gate_body(sq_ref, sk_ref, tq_ref, tk_ref, q_ref, k_ref, v_ref,
                    o_ref, g_ref):
    compat = _dists(sq_ref[0], sk_ref[0]) - _dists(tq_ref[0], tk_ref[0])
    gate16 = jnp.maximum(1.0 - compat * compat * GK, 0.0).astype(BF)
    g_ref[0] = gate16
    _softmax_message(gate16, q_ref[0], k_ref[0], v_ref[0], o_ref)


def _attn_reuse_body(g_in_ref, q_ref, k_ref, v_ref, o_ref):
    _softmax_message(g_in_ref[0], q_ref[0], k_ref[0], v_ref[0], o_ref)


def _fc_stack(msg_ref, feat_ref, w1_ref, b1_ref, g1_ref, be1_ref,
              w2_ref, b2_ref, g2_ref, be2_ref, w3_ref, b3_ref):
    m1 = _bn_relu(_dense_raw(msg_ref[...], w1_ref, b1_ref), g1_ref, be1_ref)
    m2 = _bn_relu(_dense_raw(m1, w2_ref, b2_ref), g2_ref, be2_ref)
    return feat_ref[...] + _dense_raw(m2, w3_ref, b3_ref)


def _fc_proj_body(msg_ref, feat_ref,
                  w1_ref, b1_ref, g1_ref, be1_ref,
                  w2_ref, b2_ref, g2_ref, be2_ref, w3_ref, b3_ref,
                  wc_ref, bc_ref, gc_ref, bec_ref,
                  wq_ref, bq_ref, wk_ref, bk_ref, wv_ref, bv_ref,
                  feat_out_ref, q_ref, k_ref, v_ref):
    res = _fc_stack(msg_ref, feat_ref, w1_ref, b1_ref, g1_ref, be1_ref,
                    w2_ref, b2_ref, g2_ref, be2_ref, w3_ref, b3_ref)
    feat = _bn_relu(_dense_raw(res, wc_ref, bc_ref), gc_ref, bec_ref)
    feat_out_ref[...] = feat
    _qkv_out(feat, wq_ref, bq_ref, wk_ref, bk_ref, wv_ref, bv_ref, q_ref, k_ref, v_ref)


def _fc_body(msg_ref, feat_ref,
             w1_ref, b1_ref, g1_ref, be1_ref,
             w2_ref, b2_ref, g2_ref, be2_ref, w3_ref, b3_ref, out_ref):
    out_ref[...] = _fc_stack(msg_ref, feat_ref, w1_ref, b1_ref, g1_ref, be1_ref,
                             w2_ref, b2_ref, g2_ref, be2_ref, w3_ref, b3_ref)


def _row(v):
    return v.reshape(1, -1)


def _attention0(q, k, v, src_p, tgt_p, bs, n, tq):
    qs_c = pl.BlockSpec((1, tq, 8), lambda b, i: (b, i, 0))
    ks_c = pl.BlockSpec((1, n, 8), lambda b, i: (b, 0, 0))
    msg, gate = pl.pallas_call(
        _attn_gate_body,
        out_shape=(jax.ShapeDtypeStruct((bs, n, C), BF),
                   jax.ShapeDtypeStruct((bs, n, n), BF)),
        grid=(bs, n // tq),
        in_specs=[qs_c, ks_c, qs_c, ks_c,
                  pl.BlockSpec((1, tq, C), lambda b, i: (b, i, 0)),
                  pl.BlockSpec((1, n, C), lambda b, i: (b, 0, 0)),
                  pl.BlockSpec((1, n, C), lambda b, i: (b, 0, 0))],
        out_specs=(pl.BlockSpec((1, tq, C), lambda b, i: (b, i, 0)),
                   pl.BlockSpec((1, tq, n), lambda b, i: (b, i, 0))),
        compiler_params=pltpu.CompilerParams(
            dimension_semantics=("parallel", "parallel"),
            vmem_limit_bytes=64 << 20),
    )(src_p, src_p, tgt_p, tgt_p, q, k, v)
    return msg.reshape(bs * n, C), gate


def _attention1(q, k, v, gate, bs, n, tq):
    msg = pl.pallas_call(
        _attn_reuse_body,
        out_shape=jax.ShapeDtypeStruct((bs, n, C), BF),
        grid=(bs, n // tq),
        in_specs=[pl.BlockSpec((1, tq, n), lambda b, i: (b, i, 0)),
                  pl.BlockSpec((1, tq, C), lambda b, i: (b, i, 0)),
                  pl.BlockSpec((1, n, C), lambda b, i: (b, 0, 0)),
                  pl.BlockSpec((1, n, C), lambda b, i: (b, 0, 0))],
        out_specs=pl.BlockSpec((1, tq, C), lambda b, i: (b, i, 0)),
        compiler_params=pltpu.CompilerParams(
            dimension_semantics=("parallel", "parallel"),
            vmem_limit_bytes=64 << 20),
    )(gate, q, k, v)
    return msg.reshape(bs * n, C)


def kernel(w0, b0,
           l0_wc, l0_bc, l0_gc, l0_bec,
           l0_wq, l0_bq, l0_wk, l0_bk, l0_wv, l0_bv,
           l0_w1, l0_b1, l0_g1, l0_be1, l0_w2, l0_b2, l0_g2, l0_be2, l0_w3, l0_b3,
           l1_wc, l1_bc, l1_gc, l1_bec,
           l1_wq, l1_bq, l1_wk, l1_bk, l1_wv, l1_bv,
           l1_w1, l1_b1, l1_g1, l1_be1, l1_w2, l1_b2, l1_g2, l1_be2, l1_w3, l1_b3,
           src_keypts, tgt_keypts_all):
    bs, n, _ = src_keypts.shape
    m = bs * n
    tq = _tile(n)
    tgt = jnp.mean(tgt_keypts_all, axis=2)
    src_p = jnp.pad(src_keypts, ((0, 0), (0, 0), (0, 5)))
    tgt_p = jnp.pad(tgt, ((0, 0), (0, 0), (0, 5)))
    corr = jnp.concatenate([src_keypts, tgt], axis=-1)
    corr = corr - jnp.mean(corr, axis=1, keepdims=True)
    x = jnp.pad(corr.reshape(m, 6), ((0, 0), (0, 2)))
    w0p = jnp.pad(w0, ((0, 0), (0, 2)))

    s3 = (bs, n, C)
    fshape = jax.ShapeDtypeStruct((m, C), F32)
    bshape = jax.ShapeDtypeStruct((m, C), BF)

    feat, q, k, v = pl.pallas_call(
        _proj0_body,
        out_shape=(fshape, bshape, bshape, bshape),
        compiler_params=pltpu.CompilerParams(vmem_limit_bytes=64 << 20),
    )(x, w0p, _row(b0), l0_wc, _row(l0_bc), _row(l0_gc), _row(l0_bec),
      l0_wq, _row(l0_bq), l0_wk, _row(l0_bk), l0_wv, _row(l0_bv))
    msg, gate = _attention0(q.reshape(s3), k.reshape(s3), v.reshape(s3),
                            src_p, tgt_p, bs, n, tq)

    feat, q, k, v = pl.pallas_call(
        _fc_proj_body,
        out_shape=(fshape, bshape, bshape, bshape),
        compiler_params=pltpu.CompilerParams(vmem_limit_bytes=96 << 20),
    )(msg, feat,
      l0_w1, _row(l0_b1), _row(l0_g1), _row(l0_be1),
      l0_w2, _row(l0_b2), _row(l0_g2), _row(l0_be2), l0_w3, _row(l0_b3),
      l1_wc, _row(l1_bc), _row(l1_gc), _row(l1_bec),
      l1_wq, _row(l1_bq), l1_wk, _row(l1_bk), l1_wv, _row(l1_bv))
    msg = _attention1(q.reshape(s3), k.reshape(s3), v.reshape(s3), gate, bs, n, tq)

    out = pl.pallas_call(
        _fc_body,
        out_shape=fshape,
        compiler_params=pltpu.CompilerParams(vmem_limit_bytes=64 << 20),
    )(msg, feat,
      l1_w1, _row(l1_b1), _row(l1_g1), _row(l1_be1),
      l1_w2, _row(l1_b2), _row(l1_g2), _row(l1_be2), l1_w3, _row(l1_b3))

    return jnp.transpose(out.reshape(bs, n, C), (0, 2, 1))


# speedup vs baseline: 1.2851x; 1.0331x over previous
import jax
import jax.numpy as jnp
from jax.experimental import pallas as pl
from jax.experimental.pallas import tpu as pltpu

C = 128
EPS = 1e-5
GK = 1.0 / 1.2 ** 2
INV = 1.0 / float(C) ** 0.5
BF = jnp.bfloat16
F32 = jnp.float32


def _tile(n, cap=512):
    for t in (cap, 256, 128, 64, 32, 16):
        if t <= n and n % t == 0:
            return t
    return n


def _dnt(a, b):
    return jax.lax.dot_general(a, b, (((1,), (1,)), ((), ())),
                               preferred_element_type=F32)


def _bn_relu(y, g_ref, b_ref):
    m = y.shape[0]
    ones = jnp.ones((1, m), F32)
    inv_m = 1.0 / m
    mu = jnp.dot(ones, y, preferred_element_type=F32) * inv_m
    d = y - mu
    var = jnp.dot(ones, d * d, preferred_element_type=F32) * inv_m
    return jnp.maximum(d * (g_ref[...] * jax.lax.rsqrt(var + EPS)) + b_ref[...], 0.0)


def _dense_raw(a, w_ref, b_ref):
    return _dnt(a.astype(BF), w_ref[...].astype(BF)) + b_ref[...]


def _qkv_out(feat, wq_ref, bq_ref, wk_ref, bk_ref, wv_ref, bv_ref,
             q_ref, k_ref, v_ref):
    f16 = feat.astype(BF)
    q_ref[...] = (_dnt(f16, (wq_ref[...] * INV).astype(BF))
                  + bq_ref[...] * INV).astype(BF)
    k_ref[...] = (_dnt(f16, wk_ref[...].astype(BF)) + bk_ref[...]).astype(BF)
    v_ref[...] = (_dnt(f16, wv_ref[...].astype(BF)) + bv_ref[...]).astype(BF)


def _proj0_body(x_ref, w0p_ref, b0_ref, wc_ref, bc_ref, gc_ref, bec_ref,
                wq_ref, bq_ref, wk_ref, bk_ref, wv_ref, bv_ref,
                feat_ref, q_ref, k_ref, v_ref):
    fold = jnp.dot(wc_ref[...], w0p_ref[...], preferred_element_type=F32)
    bias = _dnt(b0_ref[...], wc_ref[...]) + bc_ref[...]
    y = _dnt(x_ref[...].astype(BF), fold.astype(BF)) + bias
    feat = _bn_relu(y, gc_ref, bec_ref)
    feat_ref[...] = feat
    _qkv_out(feat, wq_ref, bq_ref, wk_ref, bk_ref, wv_ref, bv_ref, q_ref, k_ref, v_ref)


def _dists(aq, ak):
    inner = _dnt(aq, ak)
    rq = jnp.sum(aq * aq, axis=-1, keepdims=True)
    ones = jnp.ones((1, ak.shape[-1]), F32)
    rk = _dnt(ones, ak * ak)
    return jnp.sqrt(jnp.maximum(rq + rk - 2.0 * inner, 0.0))


def _softmax_message(gate16, q, k, v, o_ref):
    logits = gate16.astype(F32) * _dnt(q, k)
    mx = jnp.max(logits, axis=-1, keepdims=True)
    e = jnp.exp(logits - mx)
    w = e * pl.reciprocal(jnp.sum(e, axis=-1, keepdims=True), approx=True)
    o_ref[0] = jnp.dot(w.astype(BF), v, preferred_element_type=F32).astype(o_ref.dtype)


def _attn_gate_body(sq_ref, sk_ref, tq_ref, tk_ref, q_ref, k_ref, v_ref,
                    o_ref, g_ref):
    compat = _dists(sq_ref[0], sk_ref[0]) - _dists(tq_ref[0], tk_ref[0])
    gate16 = jnp.maximum(1.0 - compat * compat * GK, 0.0).astype(BF)
    g_ref[0] = gate16
    _softmax_message(gate16, q_ref[0], k_ref[0], v_ref[0], o_ref)


def _attn_reuse_body(g_in_ref, q_ref, k_ref, v_ref, o_ref):
    _softmax_message(g_in_ref[0], q_ref[0], k_ref[0], v_ref[0], o_ref)


def _fc_stack(msg_ref, feat_ref, w1_ref, b1_ref, g1_ref, be1_ref,
              w2_ref, b2_ref, g2_ref, be2_ref, w3_ref, b3_ref):
    m1 = _bn_relu(_dense_raw(msg_ref[...], w1_ref, b1_ref), g1_ref, be1_ref)
    m2 = _bn_relu(_dense_raw(m1, w2_ref, b2_ref), g2_ref, be2_ref)
    return feat_ref[...] + _dense_raw(m2, w3_ref, b3_ref)


def _fc_proj_body(msg_ref, feat_ref,
                  w1_ref, b1_ref, g1_ref, be1_ref,
                  w2_ref, b2_ref, g2_ref, be2_ref, w3_ref, b3_ref,
                  wc_ref, bc_ref, gc_ref, bec_ref,
                  wq_ref, bq_ref, wk_ref, bk_ref, wv_ref, bv_ref,
                  feat_out_ref, q_ref, k_ref, v_ref):
    res = _fc_stack(msg_ref, feat_ref, w1_ref, b1_ref, g1_ref, be1_ref,
                    w2_ref, b2_ref, g2_ref, be2_ref, w3_ref, b3_ref)
    feat = _bn_relu(_dense_raw(res, wc_ref, bc_ref), gc_ref, bec_ref)
    feat_out_ref[...] = feat
    _qkv_out(feat, wq_ref, bq_ref, wk_ref, bk_ref, wv_ref, bv_ref, q_ref, k_ref, v_ref)


def _make_fc_out_body(bs, n):
    def body(msg_ref, feat_ref,
             w1_ref, b1_ref, g1_ref, be1_ref,
             w2_ref, b2_ref, g2_ref, be2_ref, w3_ref, b3_ref, out_ref):
        res = _fc_stack(msg_ref, feat_ref, w1_ref, b1_ref, g1_ref, be1_ref,
                        w2_ref, b2_ref, g2_ref, be2_ref, w3_ref, b3_ref)
        for b in range(bs):
            out_ref[b] = res[b * n:(b + 1) * n].T
    return body


def _row(v):
    return v.reshape(1, -1)


def _attention0(q, k, v, src_p, tgt_p, bs, n, tq):
    qs_c = pl.BlockSpec((1, tq, 8), lambda b, i: (b, i, 0))
    ks_c = pl.BlockSpec((1, n, 8), lambda b, i: (b, 0, 0))
    msg, gate = pl.pallas_call(
        _attn_gate_body,
        out_shape=(jax.ShapeDtypeStruct((bs, n, C), BF),
                   jax.ShapeDtypeStruct((bs, n, n), BF)),
        grid=(bs, n // tq),
        in_specs=[qs_c, ks_c, qs_c, ks_c,
                  pl.BlockSpec((1, tq, C), lambda b, i: (b, i, 0)),
                  pl.BlockSpec((1, n, C), lambda b, i: (b, 0, 0)),
                  pl.BlockSpec((1, n, C), lambda b, i: (b, 0, 0))],
        out_specs=(pl.BlockSpec((1, tq, C), lambda b, i: (b, i, 0)),
                   pl.BlockSpec((1, tq, n), lambda b, i: (b, i, 0))),
        compiler_params=pltpu.CompilerParams(
            dimension_semantics=("parallel", "parallel"),
            vmem_limit_bytes=64 << 20),
    )(src_p, src_p, tgt_p, tgt_p, q, k, v)
    return msg.reshape(bs * n, C), gate


def _attention1(q, k, v, gate, bs, n, tq):
    msg = pl.pallas_call(
        _attn_reuse_body,
        out_shape=jax.ShapeDtypeStruct((bs, n, C), BF),
        grid=(bs, n // tq),
        in_specs=[pl.BlockSpec((1, tq, n), lambda b, i: (b, i, 0)),
                  pl.BlockSpec((1, tq, C), lambda b, i: (b, i, 0)),
                  pl.BlockSpec((1, n, C), lambda b, i: (b, 0, 0)),
                  pl.BlockSpec((1, n, C), lambda b, i: (b, 0, 0))],
        out_specs=pl.BlockSpec((1, tq, C), lambda b, i: (b, i, 0)),
        compiler_params=pltpu.CompilerParams(
            dimension_semantics=("parallel", "parallel"),
            vmem_limit_bytes=64 << 20),
    )(gate, q, k, v)
    return msg.reshape(bs * n, C)


def kernel(w0, b0,
           l0_wc, l0_bc, l0_gc, l0_bec,
           l0_wq, l0_bq, l0_wk, l0_bk, l0_wv, l0_bv,
           l0_w1, l0_b1, l0_g1, l0_be1, l0_w2, l0_b2, l0_g2, l0_be2, l0_w3, l0_b3,
           l1_wc, l1_bc, l1_gc, l1_bec,
           l1_wq, l1_bq, l1_wk, l1_bk, l1_wv, l1_bv,
           l1_w1, l1_b1, l1_g1, l1_be1, l1_w2, l1_b2, l1_g2, l1_be2, l1_w3, l1_b3,
           src_keypts, tgt_keypts_all):
    bs, n, _ = src_keypts.shape
    m = bs * n
    tq = _tile(n)
    tgt = jnp.mean(tgt_keypts_all, axis=2)
    src_p = jnp.pad(src_keypts, ((0, 0), (0, 0), (0, 5)))
    tgt_p = jnp.pad(tgt, ((0, 0), (0, 0), (0, 5)))
    corr = jnp.concatenate([src_keypts, tgt], axis=-1)
    corr = corr - jnp.mean(corr, axis=1, keepdims=True)
    x = jnp.pad(corr.reshape(m, 6), ((0, 0), (0, 2)))
    w0p = jnp.pad(w0, ((0, 0), (0, 2)))

    s3 = (bs, n, C)
    fshape = jax.ShapeDtypeStruct((m, C), F32)
    bshape = jax.ShapeDtypeStruct((m, C), BF)

    feat, q, k, v = pl.pallas_call(
        _proj0_body,
        out_shape=(fshape, bshape, bshape, bshape),
        compiler_params=pltpu.CompilerParams(vmem_limit_bytes=64 << 20),
    )(x, w0p, _row(b0), l0_wc, _row(l0_bc), _row(l0_gc), _row(l0_bec),
      l0_wq, _row(l0_bq), l0_wk, _row(l0_bk), l0_wv, _row(l0_bv))
    msg, gate = _attention0(q.reshape(s3), k.reshape(s3), v.reshape(s3),
                            src_p, tgt_p, bs, n, tq)

    feat, q, k, v = pl.pallas_call(
        _fc_proj_body,
        out_shape=(fshape, bshape, bshape, bshape),
        compiler_params=pltpu.CompilerParams(vmem_limit_bytes=96 << 20),
    )(msg, feat,
      l0_w1, _row(l0_b1), _row(l0_g1), _row(l0_be1),
      l0_w2, _row(l0_b2), _row(l0_g2), _row(l0_be2), l0_w3, _row(l0_b3),
      l1_wc, _row(l1_bc), _row(l1_gc), _row(l1_bec),
      l1_wq, _row(l1_bq), l1_wk, _row(l1_bk), l1_wv, _row(l1_bv))
    msg = _attention1(q.reshape(s3), k.reshape(s3), v.reshape(s3), gate, bs, n, tq)

    return pl.pallas_call(
        _make_fc_out_body(bs, n),
        out_shape=jax.ShapeDtypeStruct((bs, C, n), F32),
        compiler_params=pltpu.CompilerParams(vmem_limit_bytes=64 << 20),
    )(msg, feat,
      l1_w1, _row(l1_b1), _row(l1_g1), _row(l1_be1),
      l1_w2, _row(l1_b2), _row(l1_g2), _row(l1_be2), l1_w3, _row(l1_b3))


# speedup vs baseline: 1.3079x; 1.0177x over previous
import jax
import jax.numpy as jnp
from jax.experimental import pallas as pl
from jax.experimental.pallas import tpu as pltpu

C = 128
EPS = 1e-5
GK = 1.0 / 1.2 ** 2
INV = 1.0 / float(C) ** 0.5
BF = jnp.bfloat16
F32 = jnp.float32


def _tile(n, cap=1024):
    for t in (cap, 512, 256, 128, 64, 32, 16):
        if t <= n and n % t == 0:
            return t
    return n


def _dnt(a, b):
    return jax.lax.dot_general(a, b, (((1,), (1,)), ((), ())),
                               preferred_element_type=F32)


def _bn_relu(y, g_ref, b_ref):
    m = y.shape[0]
    ones = jnp.ones((1, m), F32)
    inv_m = 1.0 / m
    mu = jnp.dot(ones, y, preferred_element_type=F32) * inv_m
    d = y - mu
    var = jnp.dot(ones, d * d, preferred_element_type=F32) * inv_m
    return jnp.maximum(d * (g_ref[...] * jax.lax.rsqrt(var + EPS)) + b_ref[...], 0.0)


def _dense_raw(a, w_ref, b_ref):
    return _dnt(a.astype(BF), w_ref[...].astype(BF)) + b_ref[...]


def _qkv_out(feat, wq_ref, bq_ref, wk_ref, bk_ref, wv_ref, bv_ref,
             q_ref, k_ref, v_ref):
    f16 = feat.astype(BF)
    q_ref[...] = (_dnt(f16, (wq_ref[...] * INV).astype(BF))
                  + bq_ref[...] * INV).astype(BF)
    k_ref[...] = (_dnt(f16, wk_ref[...].astype(BF)) + bk_ref[...]).astype(BF)
    v_ref[...] = (_dnt(f16, wv_ref[...].astype(BF)) + bv_ref[...]).astype(BF)


def _proj0_body(x_ref, w0p_ref, b0_ref, wc_ref, bc_ref, gc_ref, bec_ref,
                wq_ref, bq_ref, wk_ref, bk_ref, wv_ref, bv_ref,
                feat_ref, q_ref, k_ref, v_ref):
    fold = jnp.dot(wc_ref[...], w0p_ref[...], preferred_element_type=F32)
    bias = _dnt(b0_ref[...], wc_ref[...]) + bc_ref[...]
    y = _dnt(x_ref[...].astype(BF), fold.astype(BF)) + bias
    feat = _bn_relu(y, gc_ref, bec_ref)
    feat_ref[...] = feat
    _qkv_out(feat, wq_ref, bq_ref, wk_ref, bk_ref, wv_ref, bv_ref, q_ref, k_ref, v_ref)


def _dists(aq, ak):
    inner = _dnt(aq, ak)
    rq = jnp.sum(aq * aq, axis=-1, keepdims=True)
    ones = jnp.ones((1, ak.shape[-1]), F32)
    rk = _dnt(ones, ak * ak)
    return jnp.sqrt(jnp.maximum(rq + rk - 2.0 * inner, 0.0))


def _softmax_message(gate16, q, k, v, o_ref):
    logits = gate16.astype(F32) * _dnt(q, k)
    mx = jnp.max(logits, axis=-1, keepdims=True)
    e = jnp.exp(logits - mx)
    w = e * pl.reciprocal(jnp.sum(e, axis=-1, keepdims=True), approx=True)
    o_ref[0] = jnp.dot(w.astype(BF), v, preferred_element_type=F32).astype(o_ref.dtype)


def _attn_gate_body(sq_ref, sk_ref, tq_ref, tk_ref, q_ref, k_ref, v_ref,
                    o_ref, g_ref):
    compat = _dists(sq_ref[0], sk_ref[0]) - _dists(tq_ref[0], tk_ref[0])
    gate16 = jnp.maximum(1.0 - compat * compat * GK, 0.0).astype(BF)
    g_ref[0] = gate16
    _softmax_message(gate16, q_ref[0], k_ref[0], v_ref[0], o_ref)


def _attn_reuse_body(g_in_ref, q_ref, k_ref, v_ref, o_ref):
    _softmax_message(g_in_ref[0], q_ref[0], k_ref[0], v_ref[0], o_ref)


def _fc_stack(msg_ref, feat_ref, w1_ref, b1_ref, g1_ref, be1_ref,
              w2_ref, b2_ref, g2_ref, be2_ref, w3_ref, b3_ref):
    m1 = _bn_relu(_dense_raw(msg_ref[...], w1_ref, b1_ref), g1_ref, be1_ref)
    m2 = _bn_relu(_dense_raw(m1, w2_ref, b2_ref), g2_ref, be2_ref)
    return feat_ref[...] + _dense_raw(m2, w3_ref, b3_ref)


def _fc_proj_body(msg_ref, feat_ref,
                  w1_ref, b1_ref, g1_ref, be1_ref,
                  w2_ref, b2_ref, g2_ref, be2_ref, w3_ref, b3_ref,
                  wc_ref, bc_ref, gc_ref, bec_ref,
                  wq_ref, bq_ref, wk_ref, bk_ref, wv_ref, bv_ref,
                  feat_out_ref, q_ref, k_ref, v_ref):
    res = _fc_stack(msg_ref, feat_ref, w1_ref, b1_ref, g1_ref, be1_ref,
                    w2_ref, b2_ref, g2_ref, be2_ref, w3_ref, b3_ref)
    feat = _bn_relu(_dense_raw(res, wc_ref, bc_ref), gc_ref, bec_ref)
    feat_out_ref[...] = feat
    _qkv_out(feat, wq_ref, bq_ref, wk_ref, bk_ref, wv_ref, bv_ref, q_ref, k_ref, v_ref)


def _make_fc_out_body(bs, n):
    def body(msg_ref, feat_ref,
             w1_ref, b1_ref, g1_ref, be1_ref,
             w2_ref, b2_ref, g2_ref, be2_ref, w3_ref, b3_ref, out_ref):
        res = _fc_stack(msg_ref, feat_ref, w1_ref, b1_ref, g1_ref, be1_ref,
                        w2_ref, b2_ref, g2_ref, be2_ref, w3_ref, b3_ref)
        for b in range(bs):
            out_ref[b] = res[b * n:(b + 1) * n].T
    return body


def _row(v):
    return v.reshape(1, -1)


def _attention0(q, k, v, src_p, tgt_p, bs, n, tq):
    qs_c = pl.BlockSpec((1, tq, 8), lambda b, i: (b, i, 0))
    ks_c = pl.BlockSpec((1, n, 8), lambda b, i: (b, 0, 0))
    msg, gate = pl.pallas_call(
        _attn_gate_body,
        out_shape=(jax.ShapeDtypeStruct((bs, n, C), BF),
                   jax.ShapeDtypeStruct((bs, n, n), BF)),
        grid=(bs, n // tq),
        in_specs=[qs_c, ks_c, qs_c, ks_c,
                  pl.BlockSpec((1, tq, C), lambda b, i: (b, i, 0)),
                  pl.BlockSpec((1, n, C), lambda b, i: (b, 0, 0)),
                  pl.BlockSpec((1, n, C), lambda b, i: (b, 0, 0))],
        out_specs=(pl.BlockSpec((1, tq, C), lambda b, i: (b, i, 0)),
                   pl.BlockSpec((1, tq, n), lambda b, i: (b, i, 0))),
        compiler_params=pltpu.CompilerParams(
            dimension_semantics=("parallel", "parallel"),
            vmem_limit_bytes=64 << 20),
    )(src_p, src_p, tgt_p, tgt_p, q, k, v)
    return msg.reshape(bs * n, C), gate


def _attention1(q, k, v, gate, bs, n, tq):
    msg = pl.pallas_call(
        _attn_reuse_body,
        out_shape=jax.ShapeDtypeStruct((bs, n, C), BF),
        grid=(bs, n // tq),
        in_specs=[pl.BlockSpec((1, tq, n), lambda b, i: (b, i, 0)),
                  pl.BlockSpec((1, tq, C), lambda b, i: (b, i, 0)),
                  pl.BlockSpec((1, n, C), lambda b, i: (b, 0, 0)),
                  pl.BlockSpec((1, n, C), lambda b, i: (b, 0, 0))],
        out_specs=pl.BlockSpec((1, tq, C), lambda b, i: (b, i, 0)),
        compiler_params=pltpu.CompilerParams(
            dimension_semantics=("parallel", "parallel"),
            vmem_limit_bytes=64 << 20),
    )(gate, q, k, v)
    return msg.reshape(bs * n, C)


def kernel(w0, b0,
           l0_wc, l0_bc, l0_gc, l0_bec,
           l0_wq, l0_bq, l0_wk, l0_bk, l0_wv, l0_bv,
           l0_w1, l0_b1, l0_g1, l0_be1, l0_w2, l0_b2, l0_g2, l0_be2, l0_w3, l0_b3,
           l1_wc, l1_bc, l1_gc, l1_bec,
           l1_wq, l1_bq, l1_wk, l1_bk, l1_wv, l1_bv,
           l1_w1, l1_b1, l1_g1, l1_be1, l1_w2, l1_b2, l1_g2, l1_be2, l1_w3, l1_b3,
           src_keypts, tgt_keypts_all):
    bs, n, _ = src_keypts.shape
    m = bs * n
    tq = _tile(n)
    tgt = jnp.mean(tgt_keypts_all, axis=2)
    src_p = jnp.pad(src_keypts, ((0, 0), (0, 0), (0, 5)))
    tgt_p = jnp.pad(tgt, ((0, 0), (0, 0), (0, 5)))
    corr = jnp.concatenate([src_keypts, tgt], axis=-1)
    corr = corr - jnp.mean(corr, axis=1, keepdims=True)
    x = jnp.pad(corr.reshape(m, 6), ((0, 0), (0, 2)))
    w0p = jnp.pad(w0, ((0, 0), (0, 2)))

    s3 = (bs, n, C)
    fshape = jax.ShapeDtypeStruct((m, C), F32)
    bshape = jax.ShapeDtypeStruct((m, C), BF)

    feat, q, k, v = pl.pallas_call(
        _proj0_body,
        out_shape=(fshape, bshape, bshape, bshape),
        compiler_params=pltpu.CompilerParams(vmem_limit_bytes=64 << 20),
    )(x, w0p, _row(b0), l0_wc, _row(l0_bc), _row(l0_gc), _row(l0_bec),
      l0_wq, _row(l0_bq), l0_wk, _row(l0_bk), l0_wv, _row(l0_bv))
    msg, gate = _attention0(q.reshape(s3), k.reshape(s3), v.reshape(s3),
                            src_p, tgt_p, bs, n, tq)

    feat, q, k, v = pl.pallas_call(
        _fc_proj_body,
        out_shape=(fshape, bshape, bshape, bshape),
        compiler_params=pltpu.CompilerParams(vmem_limit_bytes=96 << 20),
    )(msg, feat,
      l0_w1, _row(l0_b1), _row(l0_g1), _row(l0_be1),
      l0_w2, _row(l0_b2), _row(l0_g2), _row(l0_be2), l0_w3, _row(l0_b3),
      l1_wc, _row(l1_bc), _row(l1_gc), _row(l1_bec),
      l1_wq, _row(l1_bq), l1_wk, _row(l1_bk), l1_wv, _row(l1_bv))
    msg = _attention1(q.reshape(s3), k.reshape(s3), v.reshape(s3), gate, bs, n, tq)

    return pl.pallas_call(
        _make_fc_out_body(bs, n),
        out_shape=jax.ShapeDtypeStruct((bs, C, n), F32),
        compiler_params=pltpu.CompilerParams(vmem_limit_bytes=64 << 20),
    )(msg, feat,
      l1_w1, _row(l1_b1), _row(l1_g1), _row(l1_be1),
      l1_w2, _row(l1_b2), _row(l1_g2), _row(l1_be2), l1_w3, _row(l1_b3))


# speedup vs baseline: 1.3772x; 1.0530x over previous
import jax
import jax.numpy as jnp
from jax.experimental import pallas as pl
from jax.experimental.pallas import tpu as pltpu

C = 128
EPS = 1e-5
GK = 1.0 / 1.2 ** 2
INV = 1.0 / float(C) ** 0.5
BF = jnp.bfloat16
F32 = jnp.float32


def _tile(n, cap=1024):
    for t in (cap, 512, 256, 128, 64, 32, 16):
        if t <= n and n % t == 0:
            return t
    return n


def _dnt(a, b):
    return jax.lax.dot_general(a, b, (((1,), (1,)), ((), ())),
                               preferred_element_type=F32)


def _bn_relu(y, g_ref, b_ref):
    m = y.shape[0]
    ones = jnp.ones((1, m), F32)
    inv_m = 1.0 / m
    mu = jnp.dot(ones, y, preferred_element_type=F32) * inv_m
    d = y - mu
    var = jnp.dot(ones, d * d, preferred_element_type=F32) * inv_m
    return jnp.maximum(d * (g_ref[...] * jax.lax.rsqrt(var + EPS)) + b_ref[...], 0.0)


def _dense_raw(a, w_ref, b_ref):
    return _dnt(a.astype(BF), w_ref[...].astype(BF)) + b_ref[...]


def _qkv_out(feat, wq_ref, bq_ref, wk_ref, bk_ref, wv_ref, bv_ref,
             q_ref, k_ref, v_ref):
    f16 = feat.astype(BF)
    q_ref[...] = (_dnt(f16, (wq_ref[...] * INV).astype(BF))
                  + bq_ref[...] * INV).astype(BF)
    k_ref[...] = (_dnt(f16, wk_ref[...].astype(BF)) + bk_ref[...]).astype(BF)
    v_ref[...] = (_dnt(f16, wv_ref[...].astype(BF)) + bv_ref[...]).astype(BF)


def _proj0_body(x_ref, w0p_ref, b0_ref, wc_ref, bc_ref, gc_ref, bec_ref,
                wq_ref, bq_ref, wk_ref, bk_ref, wv_ref, bv_ref,
                feat_ref, q_ref, k_ref, v_ref):
    fold = jnp.dot(wc_ref[...], w0p_ref[...], preferred_element_type=F32)
    bias = _dnt(b0_ref[...], wc_ref[...]) + bc_ref[...]
    y = _dnt(x_ref[...].astype(BF), fold.astype(BF)) + bias
    feat = _bn_relu(y, gc_ref, bec_ref)
    feat_ref[...] = feat
    _qkv_out(feat, wq_ref, bq_ref, wk_ref, bk_ref, wv_ref, bv_ref, q_ref, k_ref, v_ref)


def _dists(aq, ak):
    inner = _dnt(aq, ak)
    rq = jnp.sum(aq * aq, axis=-1, keepdims=True)
    ones = jnp.ones((1, ak.shape[-1]), F32)
    rk = _dnt(ones, ak * ak)
    return jnp.sqrt(jnp.maximum(rq + rk - 2.0 * inner, 0.0))


def _softmax_message(gate16, q, k, v, o_ref):
    logits = gate16.astype(F32) * _dnt(q, k)
    mx = jnp.max(logits, axis=-1, keepdims=True)
    e = jnp.exp(logits - mx)
    w = e * pl.reciprocal(jnp.sum(e, axis=-1, keepdims=True), approx=True)
    o_ref[0] = jnp.dot(w.astype(BF), v, preferred_element_type=F32).astype(o_ref.dtype)


def _attn_gate_body(sq_ref, sk_ref, tq_ref, tk_ref, q_ref, k_ref, v_ref,
                    o_ref, g_ref):
    compat = _dists(sq_ref[0], sk_ref[0]) - _dists(tq_ref[0], tk_ref[0])
    gate16 = jnp.maximum(1.0 - compat * compat * GK, 0.0).astype(BF)
    g_ref[0] = gate16
    _softmax_message(gate16, q_ref[0], k_ref[0], v_ref[0], o_ref)


def _attn_reuse_body(g_in_ref, q_ref, k_ref, v_ref, o_ref):
    _softmax_message(g_in_ref[0], q_ref[0], k_ref[0], v_ref[0], o_ref)


def _fc_stack(msg_ref, feat_ref, w1_ref, b1_ref, g1_ref, be1_ref,
              w2_ref, b2_ref, g2_ref, be2_ref, w3_ref, b3_ref):
    m1 = _bn_relu(_dense_raw(msg_ref[...], w1_ref, b1_ref), g1_ref, be1_ref)
    m2 = _bn_relu(_dense_raw(m1, w2_ref, b2_ref), g2_ref, be2_ref)
    return feat_ref[...] + _dense_raw(m2, w3_ref, b3_ref)


def _fc_proj_body(msg_ref, feat_ref,
                  w1_ref, b1_ref, g1_ref, be1_ref,
                  w2_ref, b2_ref, g2_ref, be2_ref, w3_ref, b3_ref,
                  wc_ref, bc_ref, gc_ref, bec_ref,
                  wq_ref, bq_ref, wk_ref, bk_ref, wv_ref, bv_ref,
                  feat_out_ref, q_ref, k_ref, v_ref):
    res = _fc_stack(msg_ref, feat_ref, w1_ref, b1_ref, g1_ref, be1_ref,
                    w2_ref, b2_ref, g2_ref, be2_ref, w3_ref, b3_ref)
    feat = _bn_relu(_dense_raw(res, wc_ref, bc_ref), gc_ref, bec_ref)
    feat_out_ref[...] = feat
    _qkv_out(feat, wq_ref, bq_ref, wk_ref, bk_ref, wv_ref, bv_ref, q_ref, k_ref, v_ref)


def _make_fc_out_body(bs, n):
    def body(msg_ref, feat_ref,
             w1_ref, b1_ref, g1_ref, be1_ref,
             w2_ref, b2_ref, g2_ref, be2_ref, w3_ref, b3_ref, out_ref):
        res = _fc_stack(msg_ref, feat_ref, w1_ref, b1_ref, g1_ref, be1_ref,
                        w2_ref, b2_ref, g2_ref, be2_ref, w3_ref, b3_ref)
        for b in range(bs):
            out_ref[b] = res[b * n:(b + 1) * n].T
    return body


def _row(v):
    return v.reshape(1, -1)


def _attention0(q, k, v, src_p, tgt_p, bs, n, tq):
    qs_c = pl.BlockSpec((1, tq, 8), lambda b, i: (b, i, 0))
    ks_c = pl.BlockSpec((1, n, 8), lambda b, i: (b, 0, 0))
    msg, gate = pl.pallas_call(
        _attn_gate_body,
        out_shape=(jax.ShapeDtypeStruct((bs, n, C), BF),
                   jax.ShapeDtypeStruct((bs, n, n), BF)),
        grid=(bs, n // tq),
        in_specs=[qs_c, ks_c, qs_c, ks_c,
                  pl.BlockSpec((1, tq, C), lambda b, i: (b, i, 0)),
                  pl.BlockSpec((1, n, C), lambda b, i: (b, 0, 0)),
                  pl.BlockSpec((1, n, C), lambda b, i: (b, 0, 0))],
        out_specs=(pl.BlockSpec((1, tq, C), lambda b, i: (b, i, 0)),
                   pl.BlockSpec((1, tq, n), lambda b, i: (b, i, 0))),
        compiler_params=pltpu.CompilerParams(
            dimension_semantics=("parallel", "parallel"),
            vmem_limit_bytes=64 << 20),
    )(src_p, src_p, tgt_p, tgt_p, q, k, v)
    return msg.reshape(bs * n, C), gate


def _attention1(q, k, v, gate, bs, n, tq):
    msg = pl.pallas_call(
        _attn_reuse_body,
        out_shape=jax.ShapeDtypeStruct((bs, n, C), BF),
        grid=(bs, n // tq),
        in_specs=[pl.BlockSpec((1, tq, n), lambda b, i: (b, i, 0)),
                  pl.BlockSpec((1, tq, C), lambda b, i: (b, i, 0)),
                  pl.BlockSpec((1, n, C), lambda b, i: (b, 0, 0)),
                  pl.BlockSpec((1, n, C), lambda b, i: (b, 0, 0))],
        out_specs=pl.BlockSpec((1, tq, C), lambda b, i: (b, i, 0)),
        compiler_params=pltpu.CompilerParams(
            dimension_semantics=("parallel", "parallel"),
            vmem_limit_bytes=64 << 20),
    )(gate, q, k, v)
    return msg.reshape(bs * n, C)


def kernel(w0, b0,
           l0_wc, l0_bc, l0_gc, l0_bec,
           l0_wq, l0_bq, l0_wk, l0_bk, l0_wv, l0_bv,
           l0_w1, l0_b1, l0_g1, l0_be1, l0_w2, l0_b2, l0_g2, l0_be2, l0_w3, l0_b3,
           l1_wc, l1_bc, l1_gc, l1_bec,
           l1_wq, l1_bq, l1_wk, l1_bk, l1_wv, l1_bv,
           l1_w1, l1_b1, l1_g1, l1_be1, l1_w2, l1_b2, l1_g2, l1_be2, l1_w3, l1_b3,
           src_keypts, tgt_keypts_all):
    bs, n, _ = src_keypts.shape
    m = bs * n
    tq = _tile(n)
    tgt = jnp.mean(tgt_keypts_all, axis=2)
    src_p = jnp.pad(src_keypts, ((0, 0), (0, 0), (0, 5)))
    tgt_p = jnp.pad(tgt, ((0, 0), (0, 0), (3, 2)))
    corr8 = src_p + tgt_p
    x = (corr8 - jnp.mean(corr8, axis=1, keepdims=True)).reshape(m, 8)
    w0p = jnp.pad(w0, ((0, 0), (0, 2)))

    s3 = (bs, n, C)
    fshape = jax.ShapeDtypeStruct((m, C), F32)
    bshape = jax.ShapeDtypeStruct((m, C), BF)

    feat, q, k, v = pl.pallas_call(
        _proj0_body,
        out_shape=(fshape, bshape, bshape, bshape),
        compiler_params=pltpu.CompilerParams(vmem_limit_bytes=64 << 20),
    )(x, w0p, _row(b0), l0_wc, _row(l0_bc), _row(l0_gc), _row(l0_bec),
      l0_wq, _row(l0_bq), l0_wk, _row(l0_bk), l0_wv, _row(l0_bv))
    msg, gate = _attention0(q.reshape(s3), k.reshape(s3), v.reshape(s3),
                            src_p, tgt_p, bs, n, tq)

    feat, q, k, v = pl.pallas_call(
        _fc_proj_body,
        out_shape=(fshape, bshape, bshape, bshape),
        compiler_params=pltpu.CompilerParams(vmem_limit_bytes=96 << 20),
    )(msg, feat,
      l0_w1, _row(l0_b1), _row(l0_g1), _row(l0_be1),
      l0_w2, _row(l0_b2), _row(l0_g2), _row(l0_be2), l0_w3, _row(l0_b3),
      l1_wc, _row(l1_bc), _row(l1_gc), _row(l1_bec),
      l1_wq, _row(l1_bq), l1_wk, _row(l1_bk), l1_wv, _row(l1_bv))
    msg = _attention1(q.reshape(s3), k.reshape(s3), v.reshape(s3), gate, bs, n, tq)

    return pl.pallas_call(
        _make_fc_out_body(bs, n),
        out_shape=jax.ShapeDtypeStruct((bs, C, n), F32),
        compiler_params=pltpu.CompilerParams(vmem_limit_bytes=64 << 20),
    )(msg, feat,
      l1_w1, _row(l1_b1), _row(l1_g1), _row(l1_be1),
      l1_w2, _row(l1_b2), _row(l1_g2), _row(l1_be2), l1_w3, _row(l1_b3))


# speedup vs baseline: 1.3884x; 1.0082x over previous
import jax
import jax.numpy as jnp
from jax.experimental import pallas as pl
from jax.experimental.pallas import tpu as pltpu

C = 128
EPS = 1e-5
GK = 1.0 / 1.2 ** 2
INV = 1.0 / float(C) ** 0.5
BF = jnp.bfloat16
F32 = jnp.float32


def _tile(n, cap=1024):
    for t in (cap, 512, 256, 128, 64, 32, 16):
        if t <= n and n % t == 0:
            return t
    return n


def _dnt(a, b):
    return jax.lax.dot_general(a, b, (((1,), (1,)), ((), ())),
                               preferred_element_type=F32)


def _bn_relu(y, g_ref, b_ref):
    m = y.shape[0]
    ones = jnp.ones((1, m), F32)
    inv_m = 1.0 / m
    mu = jnp.dot(ones, y, preferred_element_type=F32) * inv_m
    d = y - mu
    var = jnp.dot(ones, d * d, preferred_element_type=F32) * inv_m
    return jnp.maximum(d * (g_ref[...] * jax.lax.rsqrt(var + EPS)) + b_ref[...], 0.0)


def _dense_raw(a, w_ref, b_ref):
    return _dnt(a.astype(BF), w_ref[...].astype(BF)) + b_ref[...]


def _qkv_out(feat, wq_ref, bq_ref, wk_ref, bk_ref, wv_ref, bv_ref,
             q_ref, k_ref, v_ref):
    f16 = feat.astype(BF)
    q_ref[...] = (_dnt(f16, (wq_ref[...] * INV).astype(BF))
                  + bq_ref[...] * INV).astype(BF)
    k_ref[...] = (_dnt(f16, wk_ref[...].astype(BF)) + bk_ref[...]).astype(BF)
    v_ref[...] = (_dnt(f16, wv_ref[...].astype(BF)) + bv_ref[...]).astype(BF)


def _make_proj0_body(bs, n):
    def body(x_ref, w0p_ref, b0_ref, wc_ref, bc_ref, gc_ref, bec_ref,
             wq_ref, bq_ref, wk_ref, bk_ref, wv_ref, bv_ref,
             feat_ref, q_ref, k_ref, v_ref):
        fold = jnp.dot(wc_ref[...], w0p_ref[...], preferred_element_type=F32)
        bias = _dnt(b0_ref[...], wc_ref[...]) + bc_ref[...]
        x3 = x_ref[...].reshape(bs, n, 8)
        xc = (x3 - jnp.mean(x3, axis=1, keepdims=True)).reshape(bs * n, 8)
        y = _dnt(xc.astype(BF), fold.astype(BF)) + bias
        feat = _bn_relu(y, gc_ref, bec_ref)
        feat_ref[...] = feat
        _qkv_out(feat, wq_ref, bq_ref, wk_ref, bk_ref, wv_ref, bv_ref, q_ref, k_ref, v_ref)
    return body


def _dists(aq, ak):
    inner = _dnt(aq, ak)
    rq = jnp.sum(aq * aq, axis=-1, keepdims=True)
    ones = jnp.ones((1, ak.shape[-1]), F32)
    rk = _dnt(ones, ak * ak)
    return jnp.sqrt(jnp.maximum(rq + rk - 2.0 * inner, 0.0))


def _softmax_message(gate16, q, k, v, o_ref):
    logits = gate16.astype(F32) * _dnt(q, k)
    mx = jnp.max(logits, axis=-1, keepdims=True)
    e = jnp.exp(logits - mx)
    w = e * pl.reciprocal(jnp.sum(e, axis=-1, keepdims=True), approx=True)
    o_ref[0] = jnp.dot(w.astype(BF), v, preferred_element_type=F32).astype(o_ref.dtype)


def _attn_gate_body(sq_ref, sk_ref, tq_ref, tk_ref, q_ref, k_ref, v_ref,
                    o_ref, g_ref):
    compat = _dists(sq_ref[0], sk_ref[0]) - _dists(tq_ref[0], tk_ref[0])
    gate16 = jnp.maximum(1.0 - compat * compat * GK, 0.0).astype(BF)
    g_ref[0] = gate16
    _softmax_message(gate16, q_ref[0], k_ref[0], v_ref[0], o_ref)


def _attn_reuse_body(g_in_ref, q_ref, k_ref, v_ref, o_ref):
    _softmax_message(g_in_ref[0], q_ref[0], k_ref[0], v_ref[0], o_ref)


def _fc_stack(msg_ref, feat_ref, w1_ref, b1_ref, g1_ref, be1_ref,
              w2_ref, b2_ref, g2_ref, be2_ref, w3_ref, b3_ref):
    m1 = _bn_relu(_dense_raw(msg_ref[...], w1_ref, b1_ref), g1_ref, be1_ref)
    m2 = _bn_relu(_dense_raw(m1, w2_ref, b2_ref), g2_ref, be2_ref)
    return feat_ref[...] + _dense_raw(m2, w3_ref, b3_ref)


def _fc_proj_body(msg_ref, feat_ref,
                  w1_ref, b1_ref, g1_ref, be1_ref,
                  w2_ref, b2_ref, g2_ref, be2_ref, w3_ref, b3_ref,
                  wc_ref, bc_ref, gc_ref, bec_ref,
                  wq_ref, bq_ref, wk_ref, bk_ref, wv_ref, bv_ref,
                  feat_out_ref, q_ref, k_ref, v_ref):
    res = _fc_stack(msg_ref, feat_ref, w1_ref, b1_ref, g1_ref, be1_ref,
                    w2_ref, b2_ref, g2_ref, be2_ref, w3_ref, b3_ref)
    feat = _bn_relu(_dense_raw(res, wc_ref, bc_ref), gc_ref, bec_ref)
    feat_out_ref[...] = feat
    _qkv_out(feat, wq_ref, bq_ref, wk_ref, bk_ref, wv_ref, bv_ref, q_ref, k_ref, v_ref)


def _make_fc_out_body(bs, n):
    def body(msg_ref, feat_ref,
             w1_ref, b1_ref, g1_ref, be1_ref,
             w2_ref, b2_ref, g2_ref, be2_ref, w3_ref, b3_ref, out_ref):
        res = _fc_stack(msg_ref, feat_ref, w1_ref, b1_ref, g1_ref, be1_ref,
                        w2_ref, b2_ref, g2_ref, be2_ref, w3_ref, b3_ref)
        for b in range(bs):
            out_ref[b] = res[b * n:(b + 1) * n].T
    return body


def _row(v):
    return v.reshape(1, -1)


def _attention0(q, k, v, src_p, tgt_p, bs, n, tq):
    qs_c = pl.BlockSpec((1, tq, 8), lambda b, i: (b, i, 0))
    ks_c = pl.BlockSpec((1, n, 8), lambda b, i: (b, 0, 0))
    msg, gate = pl.pallas_call(
        _attn_gate_body,
        out_shape=(jax.ShapeDtypeStruct((bs, n, C), BF),
                   jax.ShapeDtypeStruct((bs, n, n), BF)),
        grid=(bs, n // tq),
        in_specs=[qs_c, ks_c, qs_c, ks_c,
                  pl.BlockSpec((1, tq, C), lambda b, i: (b, i, 0)),
                  pl.BlockSpec((1, n, C), lambda b, i: (b, 0, 0)),
                  pl.BlockSpec((1, n, C), lambda b, i: (b, 0, 0))],
        out_specs=(pl.BlockSpec((1, tq, C), lambda b, i: (b, i, 0)),
                   pl.BlockSpec((1, tq, n), lambda b, i: (b, i, 0))),
        compiler_params=pltpu.CompilerParams(
            dimension_semantics=("parallel", "parallel"),
            vmem_limit_bytes=64 << 20),
    )(src_p, src_p, tgt_p, tgt_p, q, k, v)
    return msg.reshape(bs * n, C), gate


def _attention1(q, k, v, gate, bs, n, tq):
    msg = pl.pallas_call(
        _attn_reuse_body,
        out_shape=jax.ShapeDtypeStruct((bs, n, C), BF),
        grid=(bs, n // tq),
        in_specs=[pl.BlockSpec((1, tq, n), lambda b, i: (b, i, 0)),
                  pl.BlockSpec((1, tq, C), lambda b, i: (b, i, 0)),
                  pl.BlockSpec((1, n, C), lambda b, i: (b, 0, 0)),
                  pl.BlockSpec((1, n, C), lambda b, i: (b, 0, 0))],
        out_specs=pl.BlockSpec((1, tq, C), lambda b, i: (b, i, 0)),
        compiler_params=pltpu.CompilerParams(
            dimension_semantics=("parallel", "parallel"),
            vmem_limit_bytes=64 << 20),
    )(gate, q, k, v)
    return msg.reshape(bs * n, C)


def kernel(w0, b0,
           l0_wc, l0_bc, l0_gc, l0_bec,
           l0_wq, l0_bq, l0_wk, l0_bk, l0_wv, l0_bv,
           l0_w1, l0_b1, l0_g1, l0_be1, l0_w2, l0_b2, l0_g2, l0_be2, l0_w3, l0_b3,
           l1_wc, l1_bc, l1_gc, l1_bec,
           l1_wq, l1_bq, l1_wk, l1_bk, l1_wv, l1_bv,
           l1_w1, l1_b1, l1_g1, l1_be1, l1_w2, l1_b2, l1_g2, l1_be2, l1_w3, l1_b3,
           src_keypts, tgt_keypts_all):
    bs, n, _ = src_keypts.shape
    m = bs * n
    tq = _tile(n)
    tgt = jnp.mean(tgt_keypts_all, axis=2)
    src_p = jnp.pad(src_keypts, ((0, 0), (0, 0), (0, 5)))
    tgt_p = jnp.pad(tgt, ((0, 0), (0, 0), (3, 2)))
    corr8 = src_p + tgt_p
    x = corr8.reshape(m, 8)
    w0p = jnp.pad(w0, ((0, 0), (0, 2)))

    s3 = (bs, n, C)
    fshape = jax.ShapeDtypeStruct((m, C), F32)
    bshape = jax.ShapeDtypeStruct((m, C), BF)

    feat, q, k, v = pl.pallas_call(
        _make_proj0_body(bs, n),
        out_shape=(fshape, bshape, bshape, bshape),
        compiler_params=pltpu.CompilerParams(vmem_limit_bytes=64 << 20),
    )(x, w0p, _row(b0), l0_wc, _row(l0_bc), _row(l0_gc), _row(l0_bec),
      l0_wq, _row(l0_bq), l0_wk, _row(l0_bk), l0_wv, _row(l0_bv))
    msg, gate = _attention0(q.reshape(s3), k.reshape(s3), v.reshape(s3),
                            src_p, tgt_p, bs, n, tq)

    feat, q, k, v = pl.pallas_call(
        _fc_proj_body,
        out_shape=(fshape, bshape, bshape, bshape),
        compiler_params=pltpu.CompilerParams(vmem_limit_bytes=96 << 20),
    )(msg, feat,
      l0_w1, _row(l0_b1), _row(l0_g1), _row(l0_be1),
      l0_w2, _row(l0_b2), _row(l0_g2), _row(l0_be2), l0_w3, _row(l0_b3),
      l1_wc, _row(l1_bc), _row(l1_gc), _row(l1_bec),
      l1_wq, _row(l1_bq), l1_wk, _row(l1_bk), l1_wv, _row(l1_bv))
    msg = _attention1(q.reshape(s3), k.reshape(s3), v.reshape(s3), gate, bs, n, tq)

    return pl.pallas_call(
        _make_fc_out_body(bs, n),
        out_shape=jax.ShapeDtypeStruct((bs, C, n), F32),
        compiler_params=pltpu.CompilerParams(vmem_limit_bytes=64 << 20),
    )(msg, feat,
      l1_w1, _row(l1_b1), _row(l1_g1), _row(l1_be1),
      l1_w2, _row(l1_b2), _row(l1_g2), _row(l1_be2), l1_w3, _row(l1_b3))
